# Initial kernel scaffold; baseline (speedup 1.0000x reference)
import jax
import jax.numpy as jnp
from jax.experimental import pallas as pl


def kernel(x, num_tokens, emotion_embs):
    raise NotImplementedError("write your pallas kernel here")


# speedup vs baseline: 3.4883x; 3.4883x over previous
import jax
import jax.numpy as jnp
from jax.experimental import pallas as pl
from jax.experimental.pallas import tpu as pltpu

NSEG = 8
BLK = 512


def _seg_kernel(nt_ref, x_ref, e_ref, preds_ref, acc_ref):
    b = pl.program_id(0)
    j = pl.program_id(1)
    nblk = pl.num_programs(1)
    nt = nt_ref[0]
    step = nt // NSEG

    @pl.when(j == 0)
    def _init():
        acc_ref[...] = jnp.zeros_like(acc_ref)

    @pl.when(j * BLK < nt)
    def _accum():
        xb = x_ref[0]
        ss = jnp.sum(xb * xb, axis=-1, keepdims=True)
        xn = xb * jax.lax.rsqrt(ss)
        t = j * BLK + jax.lax.broadcasted_iota(jnp.int32, (NSEG, BLK), 1)
        srow = jax.lax.broadcasted_iota(jnp.int32, (NSEG, BLK), 0)
        lower = srow * step
        upper = jnp.where(srow == NSEG - 1, nt, lower + step)
        m = ((t >= lower) & (t < upper)).astype(jnp.float32)
        acc_ref[...] += jnp.dot(m, xn, preferred_element_type=jnp.float32)

    @pl.when(j == nblk - 1)
    def _classify():
        e = e_ref[...]
        en = e * jax.lax.rsqrt(jnp.sum(e * e, axis=-1, keepdims=True))
        seg = acc_ref[...]
        segn = seg * jax.lax.rsqrt(jnp.sum(seg * seg, axis=-1, keepdims=True))
        cos = jax.lax.dot_general(segn, en, (((1,), (1,)), ((), ())),
                                  preferred_element_type=jnp.float32)
        mx = jnp.max(cos, axis=-1, keepdims=True)
        idx = jax.lax.broadcasted_iota(jnp.int32, cos.shape, 1)
        pick = jnp.min(jnp.where(cos >= mx, idx, cos.shape[-1]), axis=-1)
        preds_ref[pl.ds(b, 1), :] = pick.astype(jnp.float32)[None, :]


def kernel(x, num_tokens, emotion_embs):
    B, L, D = x.shape
    E = emotion_embs.shape[0]
    nblk = L // BLK

    def x_map(b, j, nt_ref):
        last = (nt_ref[0] + BLK - 1) // BLK - 1
        return (b, jnp.minimum(j, last), 0)

    grid_spec = pltpu.PrefetchScalarGridSpec(
        num_scalar_prefetch=1,
        grid=(B, nblk),
        in_specs=[
            pl.BlockSpec((1, BLK, D), x_map),
            pl.BlockSpec((E, D), lambda b, j, nt_ref: (0, 0)),
        ],
        out_specs=pl.BlockSpec((B, NSEG), lambda b, j, nt_ref: (0, 0)),
        scratch_shapes=[pltpu.VMEM((NSEG, D), jnp.float32)],
    )
    return pl.pallas_call(
        _seg_kernel,
        grid_spec=grid_spec,
        out_shape=jax.ShapeDtypeStruct((B, NSEG), jnp.float32),
    )(num_tokens.astype(jnp.int32), x, emotion_embs)


# speedup vs baseline: 3.5019x; 1.0039x over previous
import jax
import jax.numpy as jnp
from jax.experimental import pallas as pl
from jax.experimental.pallas import tpu as pltpu

NSEG = 8
BLK = 512


def _seg_kernel(nt_ref, x_ref, e_ref, preds_ref, acc_ref):
    b = pl.program_id(0)
    j = pl.program_id(1)
    nblk = pl.num_programs(1)
    nt = nt_ref[0]
    step = nt // NSEG

    @pl.when(j == 0)
    def _init():
        acc_ref[...] = jnp.zeros_like(acc_ref)

    @pl.when(j * BLK < nt)
    def _accum():
        xb = x_ref[0]
        ss = jnp.sum(xb * xb, axis=-1)
        r = jax.lax.rsqrt(ss)
        t = j * BLK + jax.lax.broadcasted_iota(jnp.int32, (NSEG, BLK), 1)
        srow = jax.lax.broadcasted_iota(jnp.int32, (NSEG, BLK), 0)
        lower = srow * step
        upper = jnp.where(srow == NSEG - 1, nt, lower + step)
        m = ((t >= lower) & (t < upper)).astype(jnp.float32)
        msc = m * r[None, :]
        acc_ref[...] += jnp.dot(msc, xb, preferred_element_type=jnp.float32)

    @pl.when(j == nblk - 1)
    def _classify():
        e = e_ref[...]
        en = e * jax.lax.rsqrt(jnp.sum(e * e, axis=-1, keepdims=True))
        seg = acc_ref[...]
        segn = seg * jax.lax.rsqrt(jnp.sum(seg * seg, axis=-1, keepdims=True))
        cos = jax.lax.dot_general(segn, en, (((1,), (1,)), ((), ())),
                                  preferred_element_type=jnp.float32)
        mx = jnp.max(cos, axis=-1, keepdims=True)
        idx = jax.lax.broadcasted_iota(jnp.int32, cos.shape, 1)
        pick = jnp.min(jnp.where(cos >= mx, idx, cos.shape[-1]), axis=-1)
        preds_ref[pl.ds(b, 1), :] = pick.astype(jnp.float32)[None, :]


def kernel(x, num_tokens, emotion_embs):
    B, L, D = x.shape
    E = emotion_embs.shape[0]
    nblk = L // BLK

    def x_map(b, j, nt_ref):
        last = (nt_ref[0] + BLK - 1) // BLK - 1
        return (b, jnp.minimum(j, last), 0)

    grid_spec = pltpu.PrefetchScalarGridSpec(
        num_scalar_prefetch=1,
        grid=(B, nblk),
        in_specs=[
            pl.BlockSpec((1, BLK, D), x_map),
            pl.BlockSpec((E, D), lambda b, j, nt_ref: (0, 0)),
        ],
        out_specs=pl.BlockSpec((B, NSEG), lambda b, j, nt_ref: (0, 0)),
        scratch_shapes=[pltpu.VMEM((NSEG, D), jnp.float32)],
    )
    return pl.pallas_call(
        _seg_kernel,
        grid_spec=grid_spec,
        out_shape=jax.ShapeDtypeStruct((B, NSEG), jnp.float32),
    )(num_tokens.astype(jnp.int32), x, emotion_embs)


# speedup vs baseline: 5.3600x; 1.5306x over previous
import jax
import jax.numpy as jnp
from jax.experimental import pallas as pl
from jax.experimental.pallas import tpu as pltpu

NSEG = 8
BLK = 1024


def _seg_kernel(nt_ref, x_ref, e_ref, preds_ref, acc_ref):
    b = pl.program_id(0)
    j = pl.program_id(1)
    nblk = pl.num_programs(1)
    nt = nt_ref[0]
    step = nt // NSEG

    @pl.when(j == 0)
    def _init():
        acc_ref[...] = jnp.zeros_like(acc_ref)

    @pl.when(j * BLK < nt)
    def _accum():
        xb = x_ref[0]
        ss = jnp.sum(xb * xb, axis=-1, keepdims=True)
        xn = xb * jax.lax.rsqrt(ss)
        t = j * BLK + jax.lax.broadcasted_iota(jnp.int32, (NSEG, BLK), 1)
        srow = jax.lax.broadcasted_iota(jnp.int32, (NSEG, BLK), 0)
        lower = srow * step
        upper = jnp.where(srow == NSEG - 1, nt, lower + step)
        m = ((t >= lower) & (t < upper)).astype(jnp.float32)
        acc_ref[...] += jnp.dot(m, xn, preferred_element_type=jnp.float32)

    @pl.when(j == nblk - 1)
    def _classify():
        e = e_ref[...]
        en = e * jax.lax.rsqrt(jnp.sum(e * e, axis=-1, keepdims=True))
        seg = acc_ref[...]
        segn = seg * jax.lax.rsqrt(jnp.sum(seg * seg, axis=-1, keepdims=True))
        cos = jax.lax.dot_general(segn, en, (((1,), (1,)), ((), ())),
                                  preferred_element_type=jnp.float32)
        mx = jnp.max(cos, axis=-1, keepdims=True)
        idx = jax.lax.broadcasted_iota(jnp.int32, cos.shape, 1)
        pick = jnp.min(jnp.where(cos >= mx, idx, cos.shape[-1]), axis=-1)
        preds_ref[pl.ds(b, 1), :] = pick.astype(jnp.float32)[None, :]


def kernel(x, num_tokens, emotion_embs):
    B, L, D = x.shape
    E = emotion_embs.shape[0]
    nblk = L // BLK

    def x_map(b, j, nt_ref):
        last = (nt_ref[0] + BLK - 1) // BLK - 1
        return (b, jnp.minimum(j, last), 0)

    grid_spec = pltpu.PrefetchScalarGridSpec(
        num_scalar_prefetch=1,
        grid=(B, nblk),
        in_specs=[
            pl.BlockSpec((1, BLK, D), x_map),
            pl.BlockSpec((E, D), lambda b, j, nt_ref: (0, 0)),
        ],
        out_specs=pl.BlockSpec((B, NSEG), lambda b, j, nt_ref: (0, 0)),
        scratch_shapes=[pltpu.VMEM((NSEG, D), jnp.float32)],
    )
    return pl.pallas_call(
        _seg_kernel,
        grid_spec=grid_spec,
        out_shape=jax.ShapeDtypeStruct((B, NSEG), jnp.float32),
    )(num_tokens.astype(jnp.int32), x, emotion_embs)
